# baseline (speedup 1.0000x reference)
import functools

import jax
import jax.numpy as jnp
from jax import lax
from jax.experimental import pallas as pl
from jax.experimental.pallas import tpu as pltpu
from jax.experimental.pallas import tpu_sc as plsc

B, S, L = 1024, 20, 20
VOCAB, D, NCLASS = 100000, 128, 100

NIDX = S * L
CHUNK = 128
NW = 32
BPW = B // NW
ROWS_W = BPW * NIDX // CHUNK
PERIOD = 25
BPP = 8
NPER = ROWS_W // PERIOD
NBUF = 5
NVREG = D // 16
UNROLL = 4

_BOUNDARY = {}
for _m in range(1, BPP):
    _k, _off = divmod(NIDX * _m, CHUNK)
    _BOUNDARY[_k] = (_off, _m - 1)


def _sc_gather_maxpool(x2, table):
    mesh = plsc.VectorSubcoreMesh(core_axis_name="c", subcore_axis_name="s")

    @functools.partial(
        pl.kernel,
        mesh=mesh,
        out_type=jax.ShapeDtypeStruct((B, D), jnp.float32),
        scratch_types=[
            pltpu.VMEM((ROWS_W, CHUNK), jnp.int32),
            pltpu.VMEM((CHUNK, D), jnp.float32),
            pltpu.VMEM((CHUNK, D), jnp.float32),
            pltpu.VMEM((CHUNK, D), jnp.float32),
            pltpu.VMEM((CHUNK, D), jnp.float32),
            pltpu.VMEM((CHUNK, D), jnp.float32),
            pltpu.VMEM((BPW, D), jnp.float32),
            pltpu.SemaphoreType.DMA,
            pltpu.SemaphoreType.DMA,
            pltpu.SemaphoreType.DMA,
            pltpu.SemaphoreType.DMA,
            pltpu.SemaphoreType.DMA,
        ],
    )
    def k(x_hbm, table_hbm, out_hbm, idx_v, rows0, rows1, rows2, rows3,
          rows4, out_v, sem0, sem1, sem2, sem3, sem4):
        wid = lax.axis_index("s") * 2 + lax.axis_index("c")
        pltpu.sync_copy(x_hbm.at[wid], idx_v)

        rows = (rows0, rows1, rows2, rows3, rows4)
        sems = (sem0, sem1, sem2, sem3, sem4)

        for kk in range(NBUF):
            pltpu.async_copy(
                table_hbm.at[idx_v.at[kk]], rows[kk], sems[kk],
            )

        def fresh():
            return tuple(
                jnp.full((16,), -jnp.inf, jnp.float32) for _ in range(NVREG)
            )

        def reduce_rows(rref, acc, lo, hi):
            def body(r, acc):
                for u in range(UNROLL):
                    acc = tuple(
                        jnp.maximum(acc[j], rref[r * UNROLL + u, pl.ds(j * 16, 16)])
                        for j in range(NVREG)
                    )
                return acc
            return lax.fori_loop(lo // UNROLL, hi // UNROLL, body, acc)

        def period_body(p, _):
            acc = fresh()
            for kk in range(PERIOD):
                c = p * PERIOD + kk
                buf = kk % NBUF
                pltpu.make_async_copy(
                    table_hbm.at[idx_v.at[0]], rows[buf], sems[buf]
                ).wait()
                if kk in _BOUNDARY:
                    off, m = _BOUNDARY[kk]
                    acc = reduce_rows(rows[buf], acc, 0, off)
                    for j in range(NVREG):
                        out_v[p * BPP + m, pl.ds(j * 16, 16)] = acc[j]
                    acc = fresh()
                    acc = reduce_rows(rows[buf], acc, off, CHUNK)
                elif kk == PERIOD - 1:
                    acc = reduce_rows(rows[buf], acc, 0, CHUNK)
                    for j in range(NVREG):
                        out_v[p * BPP + BPP - 1, pl.ds(j * 16, 16)] = acc[j]
                else:
                    acc = reduce_rows(rows[buf], acc, 0, CHUNK)
                @pl.when(c + NBUF < ROWS_W)
                def _():
                    pltpu.async_copy(
                        table_hbm.at[idx_v.at[c + NBUF]], rows[buf], sems[buf],
                    )
            return 0

        lax.fori_loop(0, NPER, period_body, 0)
        pltpu.sync_copy(out_v, out_hbm.at[pl.ds(wid * BPW, BPW)])

    return k(x2, table)


def _fc_sigmoid(h, W, b2):
    def fc_kernel(h_ref, w_ref, b_ref, o_ref):
        acc = lax.dot_general(
            h_ref[...], w_ref[...],
            dimension_numbers=(((1,), (1,)), ((), ())),
            preferred_element_type=jnp.float32,
        )
        o_ref[...] = jax.nn.sigmoid(acc + b_ref[...])

    return pl.pallas_call(
        fc_kernel,
        out_shape=jax.ShapeDtypeStruct((B, NCLASS), jnp.float32),
    )(h, W, b2)


def kernel(x, table, W, b):
    x2 = x.astype(jnp.int32).reshape(NW, ROWS_W, CHUNK)
    h = _sc_gather_maxpool(x2, table)
    return _fc_sigmoid(h, W, b.reshape(1, NCLASS))


# speedup vs baseline: 1.0129x; 1.0129x over previous
import functools

import jax
import jax.numpy as jnp
from jax import lax
from jax.experimental import pallas as pl
from jax.experimental.pallas import tpu as pltpu
from jax.experimental.pallas import tpu_sc as plsc

B, S, L = 1024, 20, 20
VOCAB, D, NCLASS = 100000, 128, 100

NW = 32
BPW = B // NW
NVREG = D // 16
NPH = 4
RPP = S // NPH
PROWS = RPP * L
UNROLL = 4


def _sc_gather_maxpool(x, table):
    mesh = plsc.VectorSubcoreMesh(core_axis_name="c", subcore_axis_name="s")

    @functools.partial(
        pl.kernel,
        mesh=mesh,
        out_type=jax.ShapeDtypeStruct((B, D), jnp.float32),
        scratch_types=[
            pltpu.VMEM((S, L), jnp.int32),
            pltpu.VMEM((S, L), jnp.int32),
            pltpu.VMEM((PROWS, D), jnp.float32),
            pltpu.VMEM((PROWS, D), jnp.float32),
            pltpu.VMEM((PROWS, D), jnp.float32),
            pltpu.VMEM((PROWS, D), jnp.float32),
            pltpu.VMEM((BPW, D), jnp.float32),
            pltpu.SemaphoreType.DMA,
            pltpu.SemaphoreType.DMA,
            pltpu.SemaphoreType.DMA,
            pltpu.SemaphoreType.DMA,
            pltpu.SemaphoreType.DMA,
            pltpu.SemaphoreType.DMA,
        ],
    )
    def k(x_hbm, table_hbm, out_hbm, ib0, ib1, emb0, emb1, emb2, emb3,
          out_v, isem0, isem1, esem0, esem1, esem2, esem3):
        wid = lax.axis_index("s") * 2 + lax.axis_index("c")
        base = wid * BPW

        ibs = (ib0, ib1)
        isems = (isem0, isem1)
        embs = (emb0, emb1, emb2, emb3)
        esems = (esem0, esem1, esem2, esem3)

        def stage(b, i):
            pltpu.async_copy(x_hbm.at[base + b], ibs[i], isems[i])

        def stage_wait(i):
            pltpu.make_async_copy(x_hbm.at[0], ibs[i], isems[i]).wait()

        def fire(i, q):
            for s in range(RPP):
                pltpu.async_copy(
                    table_hbm.at[ibs[i].at[q * RPP + s]],
                    embs[q].at[pl.ds(s * L, L)],
                    esems[q],
                )

        def drain(q):
            for s in range(RPP):
                pltpu.make_async_copy(
                    table_hbm.at[ibs[0].at[0]],
                    embs[q].at[pl.ds(s * L, L)],
                    esems[q],
                ).wait()

        def reduce_phase(eref, acc):
            def body(r, acc):
                for u in range(UNROLL):
                    acc = tuple(
                        jnp.maximum(acc[j], eref[r * UNROLL + u, pl.ds(j * 16, 16)])
                        for j in range(NVREG)
                    )
                return acc
            return lax.fori_loop(0, PROWS // UNROLL, body, acc)

        stage(0, 0)
        stage_wait(0)
        for q in range(NPH):
            fire(0, q)
        stage(1, 1)

        def batch_body(b, i, ni):
            acc = tuple(
                jnp.full((16,), -jnp.inf, jnp.float32) for _ in range(NVREG)
            )
            for q in range(NPH):
                drain(q)
                acc = reduce_phase(embs[q], acc)
                if q == 0:
                    @pl.when(b + 1 < BPW)
                    def _():
                        stage_wait(ni)
                @pl.when(b + 1 < BPW)
                def _():
                    fire(ni, q)
            for j in range(NVREG):
                out_v[b, pl.ds(j * 16, 16)] = acc[j]
            @pl.when(b + 2 < BPW)
            def _():
                stage(b + 2, i)

        def pair_body(p, _):
            batch_body(2 * p, 0, 1)
            batch_body(2 * p + 1, 1, 0)
            return 0

        lax.fori_loop(0, BPW // 2, pair_body, 0)
        pltpu.sync_copy(out_v, out_hbm.at[pl.ds(base, BPW)])

    return k(x, table)


def _fc_sigmoid(h, W, b2):
    def fc_kernel(h_ref, w_ref, b_ref, o_ref):
        acc = lax.dot_general(
            h_ref[...], w_ref[...],
            dimension_numbers=(((1,), (1,)), ((), ())),
            preferred_element_type=jnp.float32,
        )
        o_ref[...] = jax.nn.sigmoid(acc + b_ref[...])

    return pl.pallas_call(
        fc_kernel,
        out_shape=jax.ShapeDtypeStruct((B, NCLASS), jnp.float32),
    )(h, W, b2)


def kernel(x, table, W, b):
    h = _sc_gather_maxpool(x.astype(jnp.int32), table)
    return _fc_sigmoid(h, W, b.reshape(1, NCLASS))
